# baseline (speedup 1.0000x reference)
import functools

import jax
import jax.numpy as jnp
from jax import lax
from jax.experimental import pallas as pl
from jax.experimental.pallas import tpu as pltpu
from jax.experimental.pallas import tpu_sc as plsc

_LANES = 16


def _rsqrt(x):
    xb = lax.bitcast_convert_type(x, jnp.int32)
    y = lax.bitcast_convert_type(jnp.int32(0x5F3759DF) - (xb >> 1), jnp.float32)
    for _ in range(4):
        y = y * (1.5 - 0.5 * x * y * y)
    return y


def kernel(input_token, token_table, pos_table, ln_gamma, ln_beta):
    B, L = input_token.shape
    V, H = token_table.shape
    nj = H // _LANES

    info = plsc.get_sparse_core_info()
    nw = info.num_cores * info.num_subcores
    lpw = L // nw

    mesh = plsc.VectorSubcoreMesh(core_axis_name="c", subcore_axis_name="s")

    @functools.partial(
        pl.kernel,
        out_type=jax.ShapeDtypeStruct((B, L, H), jnp.float32),
        mesh=mesh,
        compiler_params=pltpu.CompilerParams(needs_layout_passes=False),
        scratch_types=[
            pltpu.VMEM((lpw,), jnp.int32),
            pltpu.VMEM((lpw, H), jnp.float32),
            pltpu.VMEM((lpw, H), jnp.float32),
            pltpu.VMEM((H,), jnp.float32),
            pltpu.VMEM((H,), jnp.float32),
            pltpu.SemaphoreType.DMA,
        ],
    )
    def sc_kernel(tok_hbm, table_hbm, pos_hbm, g_hbm, b_hbm, out_hbm,
                  idx_v, rows_v, pos_v, g_v, bb_v, sem):
        wid = lax.axis_index("s") * info.num_cores + lax.axis_index("c")
        l0 = wid * lpw
        pltpu.sync_copy(pos_hbm.at[pl.ds(l0, lpw)], pos_v)
        pltpu.sync_copy(g_hbm, g_v)
        pltpu.sync_copy(b_hbm, bb_v)
        for b in range(B):
            pltpu.sync_copy(tok_hbm.at[b, pl.ds(l0, lpw)], idx_v)
            pltpu.async_copy(table_hbm.at[idx_v], rows_v, sem).wait()

            def row_body(l, carry):
                acc = jnp.zeros((_LANES,), jnp.float32)
                acc2 = jnp.zeros((_LANES,), jnp.float32)
                for j in range(nj):
                    sl = pl.ds(j * _LANES, _LANES)
                    v = rows_v[l, sl] + pos_v[l, sl]
                    rows_v[l, sl] = v
                    acc = acc + v
                    acc2 = acc2 + v * v
                rcp_h = jnp.float32(1.0 / H)
                mean = jnp.sum(acc) * rcp_h
                var = jnp.sum(acc2) * rcp_h - mean * mean
                r = _rsqrt(var + 1e-5)
                for j in range(nj):
                    sl = pl.ds(j * _LANES, _LANES)
                    y = (rows_v[l, sl] - mean) * r
                    rows_v[l, sl] = y * g_v[sl] + bb_v[sl]
                return carry

            lax.fori_loop(0, lpw, row_body, 0)
            pltpu.sync_copy(rows_v, out_hbm.at[b, pl.ds(l0, lpw)])

    return sc_kernel(input_token, token_table, pos_table, ln_gamma, ln_beta)


# speedup vs baseline: 1.9305x; 1.9305x over previous
import functools

import jax
import jax.numpy as jnp
from jax import lax
from jax.experimental import pallas as pl
from jax.experimental.pallas import tpu as pltpu
from jax.experimental.pallas import tpu_sc as plsc

_CHUNK = 64


def _sc_gather(idx_flat, token_table):
    N = idx_flat.shape[0]
    V, H = token_table.shape

    info = plsc.get_sparse_core_info()
    nw = info.num_cores * info.num_subcores
    rpw = N // nw
    nch = rpw // _CHUNK

    mesh = plsc.VectorSubcoreMesh(core_axis_name="c", subcore_axis_name="s")

    @functools.partial(
        pl.kernel,
        out_type=jax.ShapeDtypeStruct((N, H), jnp.float32),
        mesh=mesh,
        compiler_params=pltpu.CompilerParams(needs_layout_passes=False),
        scratch_types=[
            pltpu.VMEM((rpw,), jnp.int32),
            pltpu.VMEM((_CHUNK, H), jnp.float32),
            pltpu.VMEM((_CHUNK, H), jnp.float32),
            pltpu.SemaphoreType.DMA,
            pltpu.SemaphoreType.DMA,
        ],
    )
    def sc_kernel(idx_hbm, table_hbm, out_hbm, idx_v, buf0, buf1, sem0, sem1):
        wid = lax.axis_index("s") * info.num_cores + lax.axis_index("c")
        base = wid * rpw
        pltpu.sync_copy(idx_hbm.at[pl.ds(base, rpw)], idx_v)
        bufs = (buf0, buf1)
        sems = (sem0, sem1)
        cps = [None] * nch
        cps[0] = pltpu.async_copy(
            table_hbm.at[idx_v.at[pl.ds(0, _CHUNK)]], bufs[0], sems[0])
        for c in range(nch):
            if c + 1 < nch:
                cps[c + 1] = pltpu.async_copy(
                    table_hbm.at[idx_v.at[pl.ds((c + 1) * _CHUNK, _CHUNK)]],
                    bufs[(c + 1) % 2], sems[(c + 1) % 2])
            cps[c].wait()
            pltpu.sync_copy(bufs[c % 2],
                            out_hbm.at[pl.ds(base + c * _CHUNK, _CHUNK)])

    return sc_kernel(idx_flat, token_table)


def _tc_ln_body(tok_ref, pos_ref, g_ref, b_ref, out_ref):
    x = tok_ref[0] + pos_ref[...]
    mean = jnp.mean(x, axis=-1, keepdims=True)
    xc = x - mean
    var = jnp.mean(xc * xc, axis=-1, keepdims=True)
    y = xc * lax.rsqrt(var + 1e-5)
    out_ref[0] = y * g_ref[...] + b_ref[...]


def kernel(input_token, token_table, pos_table, ln_gamma, ln_beta):
    B, L = input_token.shape
    V, H = token_table.shape

    tok_emb = _sc_gather(input_token.reshape(-1), token_table)
    tok_emb = tok_emb.reshape(B, L, H)

    R = 256
    out = pl.pallas_call(
        _tc_ln_body,
        grid=(B, L // R),
        in_specs=[
            pl.BlockSpec((1, R, H), lambda b, i: (b, i, 0)),
            pl.BlockSpec((R, H), lambda b, i: (i, 0)),
            pl.BlockSpec((1, H), lambda b, i: (0, 0)),
            pl.BlockSpec((1, H), lambda b, i: (0, 0)),
        ],
        out_specs=pl.BlockSpec((1, R, H), lambda b, i: (b, i, 0)),
        out_shape=jax.ShapeDtypeStruct((B, L, H), jnp.float32),
    )(tok_emb, pos_table, ln_gamma.reshape(1, H), ln_beta.reshape(1, H))
    return out


# speedup vs baseline: 2.2754x; 1.1787x over previous
import functools

import jax
import jax.numpy as jnp
from jax import lax
from jax.experimental import pallas as pl
from jax.experimental.pallas import tpu as pltpu
from jax.experimental.pallas import tpu_sc as plsc

_CHUNK = 64


def _sc_gather(idx_flat, token_table):
    N = idx_flat.shape[0]
    V, H = token_table.shape

    info = plsc.get_sparse_core_info()
    nw = info.num_cores * info.num_subcores
    rpw = N // nw
    nch = rpw // _CHUNK

    mesh = plsc.VectorSubcoreMesh(core_axis_name="c", subcore_axis_name="s")

    @functools.partial(
        pl.kernel,
        out_type=jax.ShapeDtypeStruct((N, H), jnp.float32),
        mesh=mesh,
        compiler_params=pltpu.CompilerParams(needs_layout_passes=False),
        scratch_types=[
            pltpu.VMEM((rpw,), jnp.int32),
            pltpu.VMEM((_CHUNK, H), jnp.float32),
            pltpu.VMEM((_CHUNK, H), jnp.float32),
            pltpu.SemaphoreType.DMA,
            pltpu.SemaphoreType.DMA,
        ],
    )
    def sc_kernel(idx_hbm, table_hbm, out_hbm, idx_v, buf0, buf1, sem0, sem1):
        wid = lax.axis_index("s") * info.num_cores + lax.axis_index("c")
        base = wid * rpw
        pltpu.sync_copy(idx_hbm.at[pl.ds(base, rpw)], idx_v)
        bufs = (buf0, buf1)
        sems = (sem0, sem1)
        cps = [None] * nch
        cps[0] = pltpu.async_copy(
            table_hbm.at[idx_v.at[pl.ds(0, _CHUNK)]], bufs[0], sems[0])
        for c in range(nch):
            if c + 1 < nch:
                cps[c + 1] = pltpu.async_copy(
                    table_hbm.at[idx_v.at[pl.ds((c + 1) * _CHUNK, _CHUNK)]],
                    bufs[(c + 1) % 2], sems[(c + 1) % 2])
            cps[c].wait()
            pltpu.sync_copy(bufs[c % 2],
                            out_hbm.at[pl.ds(base + c * _CHUNK, _CHUNK)])

    return sc_kernel(idx_flat, token_table)


def _tc_ln_body(tok_ref, pos_ref, g_ref, b_ref, out_ref):
    x = tok_ref[0] + pos_ref[...]
    mean = jnp.mean(x, axis=-1, keepdims=True)
    xc = x - mean
    var = jnp.mean(xc * xc, axis=-1, keepdims=True)
    y = xc * lax.rsqrt(var + 1e-5)
    out_ref[0] = y * g_ref[...] + b_ref[...]


def kernel(input_token, token_table, pos_table, ln_gamma, ln_beta):
    B, L = input_token.shape
    V, H = token_table.shape

    tok_emb = _sc_gather(input_token.reshape(-1), token_table)
    tok_emb = tok_emb.reshape(B, L, H)

    R = 512
    out = pl.pallas_call(
        _tc_ln_body,
        grid=(L // R, B),
        in_specs=[
            pl.BlockSpec((1, R, H), lambda i, b: (b, i, 0)),
            pl.BlockSpec((R, H), lambda i, b: (i, 0)),
            pl.BlockSpec((1, H), lambda i, b: (0, 0)),
            pl.BlockSpec((1, H), lambda i, b: (0, 0)),
        ],
        out_specs=pl.BlockSpec((1, R, H), lambda i, b: (b, i, 0)),
        out_shape=jax.ShapeDtypeStruct((B, L, H), jnp.float32),
    )(tok_emb, pos_table, ln_gamma.reshape(1, H), ln_beta.reshape(1, H))
    return out


# speedup vs baseline: 2.4219x; 1.0644x over previous
import functools

import jax
import jax.numpy as jnp
from jax import lax
from jax.experimental import pallas as pl
from jax.experimental.pallas import tpu as pltpu
from jax.experimental.pallas import tpu_sc as plsc

_CHUNK = 64


def _sc_gather(idx_flat, token_table):
    N = idx_flat.shape[0]
    V, H = token_table.shape

    info = plsc.get_sparse_core_info()
    nw = info.num_cores * info.num_subcores
    rpw = N // nw
    nch = rpw // _CHUNK

    mesh = plsc.VectorSubcoreMesh(core_axis_name="c", subcore_axis_name="s")

    @functools.partial(
        pl.kernel,
        out_type=jax.ShapeDtypeStruct((N, H), jnp.float32),
        mesh=mesh,
        compiler_params=pltpu.CompilerParams(needs_layout_passes=False),
        scratch_types=[
            pltpu.VMEM((rpw,), jnp.int32),
            pltpu.VMEM((_CHUNK, H), jnp.float32),
            pltpu.VMEM((_CHUNK, H), jnp.float32),
            pltpu.SemaphoreType.DMA,
            pltpu.SemaphoreType.DMA,
        ],
    )
    def sc_kernel(idx_hbm, table_hbm, out_hbm, idx_v, buf0, buf1, sem0, sem1):
        wid = lax.axis_index("s") * info.num_cores + lax.axis_index("c")
        base = wid * rpw
        pltpu.sync_copy(idx_hbm.at[pl.ds(base, rpw)], idx_v)
        bufs = (buf0, buf1)
        sems = (sem0, sem1)
        cps = [None] * nch
        cps[0] = pltpu.async_copy(
            table_hbm.at[idx_v.at[pl.ds(0, _CHUNK)]], bufs[0], sems[0])
        for c in range(nch):
            if c + 1 < nch:
                cps[c + 1] = pltpu.async_copy(
                    table_hbm.at[idx_v.at[pl.ds((c + 1) * _CHUNK, _CHUNK)]],
                    bufs[(c + 1) % 2], sems[(c + 1) % 2])
            cps[c].wait()
            pltpu.sync_copy(bufs[c % 2],
                            out_hbm.at[pl.ds(base + c * _CHUNK, _CHUNK)])

    return sc_kernel(idx_flat, token_table)


def _tc_ln_body(tok_ref, pos_ref, g_ref, b_ref, out_ref):
    x = tok_ref[0] + pos_ref[...]
    mean = jnp.mean(x, axis=-1, keepdims=True)
    xc = x - mean
    var = jnp.mean(xc * xc, axis=-1, keepdims=True)
    y = xc * lax.rsqrt(var + 1e-5)
    out_ref[0] = y * g_ref[...] + b_ref[...]


def kernel(input_token, token_table, pos_table, ln_gamma, ln_beta):
    B, L = input_token.shape
    V, H = token_table.shape

    tok_emb = _sc_gather(input_token.reshape(-1), token_table)
    tok_emb = tok_emb.reshape(B, L, H)

    R = 1024
    out = pl.pallas_call(
        _tc_ln_body,
        grid=(L // R, B),
        in_specs=[
            pl.BlockSpec((1, R, H), lambda i, b: (b, i, 0)),
            pl.BlockSpec((R, H), lambda i, b: (i, 0)),
            pl.BlockSpec((1, H), lambda i, b: (0, 0)),
            pl.BlockSpec((1, H), lambda i, b: (0, 0)),
        ],
        out_specs=pl.BlockSpec((1, R, H), lambda i, b: (b, i, 0)),
        out_shape=jax.ShapeDtypeStruct((B, L, H), jnp.float32),
    )(tok_emb, pos_table, ln_gamma.reshape(1, H), ln_beta.reshape(1, H))
    return out


# speedup vs baseline: 2.4432x; 1.0088x over previous
import functools

import jax
import jax.numpy as jnp
from jax import lax
from jax.experimental import pallas as pl
from jax.experimental.pallas import tpu as pltpu
from jax.experimental.pallas import tpu_sc as plsc

_CHUNK = 64


def _sc_gather(idx_flat, token_table):
    N = idx_flat.shape[0]
    V, H = token_table.shape

    info = plsc.get_sparse_core_info()
    nw = info.num_cores * info.num_subcores
    rpw = N // nw
    nch = rpw // _CHUNK

    mesh = plsc.VectorSubcoreMesh(core_axis_name="c", subcore_axis_name="s")

    @functools.partial(
        pl.kernel,
        out_type=jax.ShapeDtypeStruct((N, H), jnp.float32),
        mesh=mesh,
        compiler_params=pltpu.CompilerParams(needs_layout_passes=False),
        scratch_types=[
            pltpu.VMEM((rpw,), jnp.int32),
            pltpu.VMEM((_CHUNK, H), jnp.float32),
            pltpu.VMEM((_CHUNK, H), jnp.float32),
            pltpu.SemaphoreType.DMA,
            pltpu.SemaphoreType.DMA,
        ],
    )
    def sc_kernel(idx_hbm, table_hbm, out_hbm, idx_v, buf0, buf1, sem0, sem1):
        wid = lax.axis_index("s") * info.num_cores + lax.axis_index("c")
        base = wid * rpw
        pltpu.sync_copy(idx_hbm.at[pl.ds(base, rpw)], idx_v)
        bufs = (buf0, buf1)
        sems = (sem0, sem1)
        cps = [None] * nch
        cps[0] = pltpu.async_copy(
            table_hbm.at[idx_v.at[pl.ds(0, _CHUNK)]], bufs[0], sems[0])
        for c in range(nch):
            if c + 1 < nch:
                cps[c + 1] = pltpu.async_copy(
                    table_hbm.at[idx_v.at[pl.ds((c + 1) * _CHUNK, _CHUNK)]],
                    bufs[(c + 1) % 2], sems[(c + 1) % 2])
            cps[c].wait()
            pltpu.sync_copy(bufs[c % 2],
                            out_hbm.at[pl.ds(base + c * _CHUNK, _CHUNK)])

    return sc_kernel(idx_flat, token_table)


def _tc_ln_body(tok_ref, pos_ref, g_ref, b_ref, out_ref):
    x = tok_ref[0] + pos_ref[...]
    mean = jnp.mean(x, axis=-1, keepdims=True)
    xc = x - mean
    var = jnp.mean(xc * xc, axis=-1, keepdims=True)
    y = xc * lax.rsqrt(var + 1e-5)
    out_ref[0] = y * g_ref[...] + b_ref[...]


def kernel(input_token, token_table, pos_table, ln_gamma, ln_beta):
    B, L = input_token.shape
    V, H = token_table.shape

    tok_emb = _sc_gather(input_token.reshape(-1), token_table)
    tok_emb = tok_emb.reshape(B, L, H)

    R = 2048
    out = pl.pallas_call(
        _tc_ln_body,
        grid=(L // R, B),
        in_specs=[
            pl.BlockSpec((1, R, H), lambda i, b: (b, i, 0)),
            pl.BlockSpec((R, H), lambda i, b: (i, 0)),
            pl.BlockSpec((1, H), lambda i, b: (0, 0)),
            pl.BlockSpec((1, H), lambda i, b: (0, 0)),
        ],
        out_specs=pl.BlockSpec((1, R, H), lambda i, b: (b, i, 0)),
        out_shape=jax.ShapeDtypeStruct((B, L, H), jnp.float32),
    )(tok_emb, pos_table, ln_gamma.reshape(1, H), ln_beta.reshape(1, H))
    return out
